# baseline (speedup 1.0000x reference)
import functools

import jax
import jax.numpy as jnp
from jax import lax
from jax.experimental import pallas as pl
from jax.experimental.pallas import tpu as pltpu
from jax.experimental.pallas import tpu_sc as plsc

N = 10000
E = 320000
DIM = 128
NC = 2
NS = 16
CH = 80
EPT = E // NS
NCHUNK = EPT // CH
HN = 5120
AGGR = HN + 8
RPT = HN // NS
ZR = 16
NBUF = 2


def _sc_agg_body(x_hbm, e_hbm, src_hbm, dst_hbm, parts_hbm,
                 si0, si1, di0, di1, g0, g1, e0, e1, d20, d21, z_v, agg_sh,
                 isem0, isem1, gsem0, gsem1, esem0, esem1, ssem0, ssem1):
    c = lax.axis_index("c")
    s = lax.axis_index("s")
    si = (si0, si1)
    di = (di0, di1)
    g = (g0, g1)
    eb = (e0, e1)
    d2 = (d20, d21)
    isem = (isem0, isem1)
    gsem = (gsem0, gsem1)
    esem = (esem0, esem1)
    ssem = (ssem0, ssem1)

    def zrow(r, carry):
        for v in range(DIM // 16):
            z_v[r, pl.ds(v * 16, 16)] = jnp.zeros((16,), jnp.float32)
        return carry
    lax.fori_loop(0, ZR, zrow, 0)
    for k in range(RPT // ZR):
        pltpu.sync_copy(z_v, agg_sh.at[pl.ds(s * RPT + k * ZR, ZR)])
    plsc.subcore_barrier()

    ebase = s * EPT
    nbase = c * HN

    def load_idx(j, b):
        pltpu.async_copy(src_hbm.at[pl.ds(ebase + j * CH, CH)],
                         si[b].at[0], isem[b])
        pltpu.async_copy(dst_hbm.at[pl.ds(ebase + j * CH, CH)],
                         di[b].at[0], isem[b])

    def wait_idx(j, b):
        pltpu.make_async_copy(src_hbm.at[pl.ds(ebase + j * CH, CH)],
                              si[b].at[0], isem[b]).wait()
        pltpu.make_async_copy(dst_hbm.at[pl.ds(ebase + j * CH, CH)],
                              di[b].at[0], isem[b]).wait()

    def load_data(j, b):
        pltpu.async_copy(x_hbm.at[si[b].at[0]], g[b], gsem[b])
        pltpu.async_copy(e_hbm.at[pl.ds(ebase + j * CH, CH)], eb[b], esem[b])

    def wait_data(j, b):
        pltpu.make_async_copy(x_hbm.at[si[b].at[0]], g[b], gsem[b]).wait()
        pltpu.make_async_copy(
            e_hbm.at[pl.ds(ebase + j * CH, CH)], eb[b], esem[b]).wait()

    def wait_scatter(b):
        pltpu.make_async_copy(g[b], agg_sh.at[d2[b].at[0]], ssem[b]).wait()

    load_idx(0, 0)
    load_idx(1, 1)
    wait_idx(0, 0)
    load_data(0, 0)

    def step(i, carry):
        j2 = i * NBUF
        for b in range(NBUF):
            j = j2 + b
            o = 1 - b
            wait_data(j, b)

            @pl.when(j + 1 < NCHUNK)
            def _():
                @pl.when(j >= 1)
                def _():
                    wait_scatter(o)
                wait_idx(j + 1, o)
                load_data(j + 1, o)

            @plsc.parallel_loop(0, CH, step=1, unroll=4)
            def crow(r):
                for v in range(DIM // 16):
                    sl = pl.ds(v * 16, 16)
                    g[b][r, sl] = jnp.maximum(g[b][r, sl] + eb[b][r, sl], 0.0)

            for v in range(CH // 16):
                sl = pl.ds(v * 16, 16)
                t = di[b][0, sl] - nbase
                ok = (t >= 0) & (t < HN)
                d2[b][0, sl] = jnp.where(ok, t, HN)

            pltpu.async_copy(g[b], agg_sh.at[d2[b].at[0]], ssem[b], add=True)

            @pl.when(j + NBUF < NCHUNK)
            def _():
                load_idx(j + NBUF, b)
        return carry
    lax.fori_loop(0, NCHUNK // NBUF, step, 0)
    for b in range(NBUF):
        wait_scatter(b)

    plsc.subcore_barrier()
    pltpu.sync_copy(agg_sh.at[pl.ds(s * RPT, RPT)],
                    parts_hbm.at[c, pl.ds(s * RPT, RPT)])


_sc_agg = pl.kernel(
    _sc_agg_body,
    out_type=jax.ShapeDtypeStruct((NC, HN, DIM), jnp.float32),
    mesh=plsc.VectorSubcoreMesh(core_axis_name="c", subcore_axis_name="s"),
    scratch_types=[
        pltpu.VMEM((8, CH), jnp.int32),
        pltpu.VMEM((8, CH), jnp.int32),
        pltpu.VMEM((8, CH), jnp.int32),
        pltpu.VMEM((8, CH), jnp.int32),
        pltpu.VMEM((CH, DIM), jnp.float32),
        pltpu.VMEM((CH, DIM), jnp.float32),
        pltpu.VMEM((CH, DIM), jnp.float32),
        pltpu.VMEM((CH, DIM), jnp.float32),
        pltpu.VMEM((8, CH), jnp.int32),
        pltpu.VMEM((8, CH), jnp.int32),
        pltpu.VMEM((ZR, DIM), jnp.float32),
        pltpu.VMEM_SHARED((AGGR, DIM), jnp.float32),
        pltpu.SemaphoreType.DMA,
        pltpu.SemaphoreType.DMA,
        pltpu.SemaphoreType.DMA,
        pltpu.SemaphoreType.DMA,
        pltpu.SemaphoreType.DMA,
        pltpu.SemaphoreType.DMA,
        pltpu.SemaphoreType.DMA,
        pltpu.SemaphoreType.DMA,
    ],
)


def _bnx_body(x_ref, xi_ref, g_ref, b_ref, o_ref):
    x = x_ref[...] * xi_ref[...]
    m = jnp.mean(x, axis=0, keepdims=True)
    v = jnp.mean(jnp.square(x - m), axis=0, keepdims=True)
    o_ref[...] = (x - m) * lax.rsqrt(v + 1e-5) * g_ref[...] + b_ref[...]


def _bnx(X, Xi, g, b):
    return pl.pallas_call(
        _bnx_body,
        out_shape=jax.ShapeDtypeStruct((N, DIM), jnp.float32),
    )(X, Xi, g.reshape(1, DIM), b.reshape(1, DIM))


def _estats_body(e_ref, o_ref):
    i = pl.program_id(0)
    x = e_ref[...]
    part = jnp.concatenate(
        [jnp.sum(x, axis=0, keepdims=True),
         jnp.sum(jnp.square(x), axis=0, keepdims=True)], axis=0)

    @pl.when(i == 0)
    def _():
        o_ref[...] = part

    @pl.when(i > 0)
    def _():
        o_ref[...] += part


def _estats(e_flat, block=8000):
    n = e_flat.shape[0]
    return pl.pallas_call(
        _estats_body,
        grid=(n // block,),
        in_specs=[pl.BlockSpec((block, DIM), lambda i: (i, 0))],
        out_specs=pl.BlockSpec((2, DIM), lambda i: (0, 0)),
        out_shape=jax.ShapeDtypeStruct((2, DIM), jnp.float32),
    )(e_flat)


def _mlp_body(x_ref, w_ref, b_ref, o_ref):
    acc = jnp.dot(x_ref[...], w_ref[...], preferred_element_type=jnp.float32)
    o_ref[...] = jnp.tanh(acc + b_ref[...])


def _mlp(x, Wt, b, block):
    n, d_in = x.shape
    d_out = Wt.shape[1]
    return pl.pallas_call(
        _mlp_body,
        grid=(n // block,),
        in_specs=[
            pl.BlockSpec((block, d_in), lambda i: (i, 0)),
            pl.BlockSpec((d_in, d_out), lambda i: (0, 0)),
            pl.BlockSpec((1, d_out), lambda i: (0, 0)),
        ],
        out_specs=pl.BlockSpec((block, d_out), lambda i: (i, 0)),
        out_shape=jax.ShapeDtypeStruct((n, d_out), jnp.float32),
    )(x, Wt, b.reshape(1, -1))


def _node_body(x_ref, p_ref, w_ref, b_ref, o_ref):
    sm = x_ref[...] + p_ref[...]
    acc = jnp.dot(sm, w_ref[...], preferred_element_type=jnp.float32)
    o_ref[...] = jnp.tanh(acc + b_ref[...])


def _node_update(x, parts_flat, W, b, block=2000):
    return pl.pallas_call(
        _node_body,
        grid=(N // block,),
        in_specs=[
            pl.BlockSpec((block, DIM), lambda i: (i, 0)),
            pl.BlockSpec((block, DIM), lambda i: (i, 0)),
            pl.BlockSpec((DIM, DIM), lambda i: (0, 0)),
            pl.BlockSpec((1, DIM), lambda i: (0, 0)),
        ],
        out_specs=pl.BlockSpec((block, DIM), lambda i: (i, 0)),
        out_shape=jax.ShapeDtypeStruct((N, DIM), jnp.float32),
    )(x, parts_flat, W.T, b.reshape(1, -1))


def kernel(X, X_importance, edge_index, edge_attr, g_in, b_in, g_e, b_e,
           We1, be1, W1, b1, We2, be2, W2, b2, We3, be3, W3, b3,
           We4, be4, W4, b4, We5, be5, W5, b5, Wfc):
    srcM = edge_index[0]
    dstM = edge_index[1]

    x = _bnx(X, X_importance, g_in, b_in)

    e_flat = edge_attr.reshape(E * 16 // DIM, DIM)
    st = _estats(e_flat)
    s8 = st[0].reshape(DIM // 16, 16).sum(axis=0)
    ss8 = st[1].reshape(DIM // 16, 16).sum(axis=0)
    mean = s8 / E
    var = ss8 / E - jnp.square(mean)
    inv = lax.rsqrt(var + 1e-5)
    scale = g_e * inv
    shift = b_e - mean * scale
    Wp = We1.T * scale[:, None]
    bp = be1 + shift @ We1.T

    e = _mlp(edge_attr, Wp, bp, block=4000)

    def gine(x, e, W, b):
        parts = _sc_agg(x, e, srcM, dstM)
        agg = parts.reshape(NC * HN, DIM)
        return _node_update(x, agg, W, b)

    x1 = gine(x, e, W1, b1)
    e = _mlp(e, We2.T, be2, block=4000)
    x2 = gine(x1, e, W2, b2)
    e = _mlp(e, We3.T, be3, block=4000)
    x3 = gine(x2, e, W3, b3)
    e = _mlp(e, We4.T, be4, block=4000)
    x4 = gine(x3, e, W4, b4)
    e = _mlp(e, We5.T, be5, block=4000)
    x5 = gine(x4, e, W5, b5)
    x6 = _mlp(x5, Wfc.T, jnp.zeros((DIM,), jnp.float32), block=2000)
    return jnp.concatenate([x1, x2, x3, x4, x5, x6], axis=-1)


# speedup vs baseline: 1.0717x; 1.0717x over previous
import functools

import jax
import jax.numpy as jnp
from jax import lax
from jax.experimental import pallas as pl
from jax.experimental.pallas import tpu as pltpu
from jax.experimental.pallas import tpu_sc as plsc

N = 10000
E = 320000
DIM = 128
NC = 2
NS = 16
CH = 80
EPT = E // NS
NCHUNK = EPT // CH
HN = 5120
AGGR = HN + 8
RPT = HN // NS
ZR = 16
NBUF = 2


def _sc_agg_body(x_hbm, e_hbm, src_hbm, dst_hbm, parts_hbm,
                 si0, si1, di0, di1, g0, g1, e0, e1, d20, d21, z_v, agg_sh,
                 isem0, isem1, gsem0, gsem1, esem0, esem1, ssem0, ssem1):
    c = lax.axis_index("c")
    s = lax.axis_index("s")
    si = (si0, si1)
    di = (di0, di1)
    g = (g0, g1)
    eb = (e0, e1)
    d2 = (d20, d21)
    isem = (isem0, isem1)
    gsem = (gsem0, gsem1)
    esem = (esem0, esem1)
    ssem = (ssem0, ssem1)

    def zrow(r, carry):
        for v in range(DIM // 16):
            z_v[r, pl.ds(v * 16, 16)] = jnp.zeros((16,), jnp.float32)
        return carry
    lax.fori_loop(0, ZR, zrow, 0)
    for k in range(RPT // ZR):
        pltpu.sync_copy(z_v, agg_sh.at[pl.ds(s * RPT + k * ZR, ZR)])
    plsc.subcore_barrier()

    ebase = s * EPT
    nbase = c * HN

    def load_idx(j, b):
        pltpu.async_copy(src_hbm.at[pl.ds(ebase + j * CH, CH)],
                         si[b].at[0], isem[b])
        pltpu.async_copy(dst_hbm.at[pl.ds(ebase + j * CH, CH)],
                         di[b].at[0], isem[b])

    def wait_idx(j, b):
        pltpu.make_async_copy(src_hbm.at[pl.ds(ebase + j * CH, CH)],
                              si[b].at[0], isem[b]).wait()
        pltpu.make_async_copy(dst_hbm.at[pl.ds(ebase + j * CH, CH)],
                              di[b].at[0], isem[b]).wait()

    def load_data(j, b):
        pltpu.async_copy(x_hbm.at[si[b].at[0]], g[b], gsem[b])
        pltpu.async_copy(e_hbm.at[pl.ds(ebase + j * CH, CH)], eb[b], esem[b])

    def wait_data(j, b):
        pltpu.make_async_copy(x_hbm.at[si[b].at[0]], g[b], gsem[b]).wait()
        pltpu.make_async_copy(
            e_hbm.at[pl.ds(ebase + j * CH, CH)], eb[b], esem[b]).wait()

    def wait_scatter(b):
        pltpu.make_async_copy(
            g[b], agg_sh.at[plsc.Indices(d2[b].at[0], ignored_value=-1)],
            ssem[b]).wait()

    load_idx(0, 0)
    load_idx(1, 1)
    wait_idx(0, 0)
    load_data(0, 0)

    def step(i, carry):
        j2 = i * NBUF
        for b in range(NBUF):
            j = j2 + b
            o = 1 - b
            wait_data(j, b)

            @pl.when(j + 1 < NCHUNK)
            def _():
                @pl.when(j >= 1)
                def _():
                    wait_scatter(o)
                wait_idx(j + 1, o)
                load_data(j + 1, o)

            @plsc.parallel_loop(0, CH, step=1, unroll=4)
            def crow(r):
                for v in range(DIM // 16):
                    sl = pl.ds(v * 16, 16)
                    g[b][r, sl] = jnp.maximum(g[b][r, sl] + eb[b][r, sl], 0.0)

            for v in range(CH // 16):
                sl = pl.ds(v * 16, 16)
                t = di[b][0, sl] - nbase
                ok = (t >= 0) & (t < HN)
                d2[b][0, sl] = jnp.where(ok, t, -1)

            pltpu.async_copy(
                g[b],
                agg_sh.at[plsc.Indices(d2[b].at[0], ignored_value=-1)],
                ssem[b], add=True)

            @pl.when(j + NBUF < NCHUNK)
            def _():
                load_idx(j + NBUF, b)
        return carry
    lax.fori_loop(0, NCHUNK // NBUF, step, 0)
    for b in range(NBUF):
        wait_scatter(b)

    plsc.subcore_barrier()
    pltpu.sync_copy(agg_sh.at[pl.ds(s * RPT, RPT)],
                    parts_hbm.at[c, pl.ds(s * RPT, RPT)])


_sc_agg = pl.kernel(
    _sc_agg_body,
    out_type=jax.ShapeDtypeStruct((NC, HN, DIM), jnp.float32),
    mesh=plsc.VectorSubcoreMesh(core_axis_name="c", subcore_axis_name="s"),
    scratch_types=[
        pltpu.VMEM((8, CH), jnp.int32),
        pltpu.VMEM((8, CH), jnp.int32),
        pltpu.VMEM((8, CH), jnp.int32),
        pltpu.VMEM((8, CH), jnp.int32),
        pltpu.VMEM((CH, DIM), jnp.float32),
        pltpu.VMEM((CH, DIM), jnp.float32),
        pltpu.VMEM((CH, DIM), jnp.float32),
        pltpu.VMEM((CH, DIM), jnp.float32),
        pltpu.VMEM((8, CH), jnp.int32),
        pltpu.VMEM((8, CH), jnp.int32),
        pltpu.VMEM((ZR, DIM), jnp.float32),
        pltpu.VMEM_SHARED((AGGR, DIM), jnp.float32),
        pltpu.SemaphoreType.DMA,
        pltpu.SemaphoreType.DMA,
        pltpu.SemaphoreType.DMA,
        pltpu.SemaphoreType.DMA,
        pltpu.SemaphoreType.DMA,
        pltpu.SemaphoreType.DMA,
        pltpu.SemaphoreType.DMA,
        pltpu.SemaphoreType.DMA,
    ],
)


def _bnx_body(x_ref, xi_ref, g_ref, b_ref, o_ref):
    x = x_ref[...] * xi_ref[...]
    m = jnp.mean(x, axis=0, keepdims=True)
    v = jnp.mean(jnp.square(x - m), axis=0, keepdims=True)
    o_ref[...] = (x - m) * lax.rsqrt(v + 1e-5) * g_ref[...] + b_ref[...]


def _bnx(X, Xi, g, b):
    return pl.pallas_call(
        _bnx_body,
        out_shape=jax.ShapeDtypeStruct((N, DIM), jnp.float32),
    )(X, Xi, g.reshape(1, DIM), b.reshape(1, DIM))


def _estats_body(e_ref, o_ref):
    i = pl.program_id(0)
    x = e_ref[...]
    part = jnp.concatenate(
        [jnp.sum(x, axis=0, keepdims=True),
         jnp.sum(jnp.square(x), axis=0, keepdims=True)], axis=0)

    @pl.when(i == 0)
    def _():
        o_ref[...] = part

    @pl.when(i > 0)
    def _():
        o_ref[...] += part


def _estats(e_flat, block=8000):
    n = e_flat.shape[0]
    return pl.pallas_call(
        _estats_body,
        grid=(n // block,),
        in_specs=[pl.BlockSpec((block, DIM), lambda i: (i, 0))],
        out_specs=pl.BlockSpec((2, DIM), lambda i: (0, 0)),
        out_shape=jax.ShapeDtypeStruct((2, DIM), jnp.float32),
    )(e_flat)


def _mlp_body(x_ref, w_ref, b_ref, o_ref):
    acc = jnp.dot(x_ref[...], w_ref[...], preferred_element_type=jnp.float32)
    o_ref[...] = jnp.tanh(acc + b_ref[...])


def _mlp(x, Wt, b, block):
    n, d_in = x.shape
    d_out = Wt.shape[1]
    return pl.pallas_call(
        _mlp_body,
        grid=(n // block,),
        in_specs=[
            pl.BlockSpec((block, d_in), lambda i: (i, 0)),
            pl.BlockSpec((d_in, d_out), lambda i: (0, 0)),
            pl.BlockSpec((1, d_out), lambda i: (0, 0)),
        ],
        out_specs=pl.BlockSpec((block, d_out), lambda i: (i, 0)),
        out_shape=jax.ShapeDtypeStruct((n, d_out), jnp.float32),
    )(x, Wt, b.reshape(1, -1))


def _node_body(x_ref, p_ref, w_ref, b_ref, o_ref):
    sm = x_ref[...] + p_ref[...]
    acc = jnp.dot(sm, w_ref[...], preferred_element_type=jnp.float32)
    o_ref[...] = jnp.tanh(acc + b_ref[...])


def _node_update(x, parts_flat, W, b, block=2000):
    return pl.pallas_call(
        _node_body,
        grid=(N // block,),
        in_specs=[
            pl.BlockSpec((block, DIM), lambda i: (i, 0)),
            pl.BlockSpec((block, DIM), lambda i: (i, 0)),
            pl.BlockSpec((DIM, DIM), lambda i: (0, 0)),
            pl.BlockSpec((1, DIM), lambda i: (0, 0)),
        ],
        out_specs=pl.BlockSpec((block, DIM), lambda i: (i, 0)),
        out_shape=jax.ShapeDtypeStruct((N, DIM), jnp.float32),
    )(x, parts_flat, W.T, b.reshape(1, -1))


def kernel(X, X_importance, edge_index, edge_attr, g_in, b_in, g_e, b_e,
           We1, be1, W1, b1, We2, be2, W2, b2, We3, be3, W3, b3,
           We4, be4, W4, b4, We5, be5, W5, b5, Wfc):
    srcM = edge_index[0]
    dstM = edge_index[1]

    x = _bnx(X, X_importance, g_in, b_in)

    e_flat = edge_attr.reshape(E * 16 // DIM, DIM)
    st = _estats(e_flat)
    s8 = st[0].reshape(DIM // 16, 16).sum(axis=0)
    ss8 = st[1].reshape(DIM // 16, 16).sum(axis=0)
    mean = s8 / E
    var = ss8 / E - jnp.square(mean)
    inv = lax.rsqrt(var + 1e-5)
    scale = g_e * inv
    shift = b_e - mean * scale
    Wp = We1.T * scale[:, None]
    bp = be1 + shift @ We1.T

    e = _mlp(edge_attr, Wp, bp, block=4000)

    def gine(x, e, W, b):
        parts = _sc_agg(x, e, srcM, dstM)
        agg = parts.reshape(NC * HN, DIM)
        return _node_update(x, agg, W, b)

    x1 = gine(x, e, W1, b1)
    e = _mlp(e, We2.T, be2, block=4000)
    x2 = gine(x1, e, W2, b2)
    e = _mlp(e, We3.T, be3, block=4000)
    x3 = gine(x2, e, W3, b3)
    e = _mlp(e, We4.T, be4, block=4000)
    x4 = gine(x3, e, W4, b4)
    e = _mlp(e, We5.T, be5, block=4000)
    x5 = gine(x4, e, W5, b5)
    x6 = _mlp(x5, Wfc.T, jnp.zeros((DIM,), jnp.float32), block=2000)
    return jnp.concatenate([x1, x2, x3, x4, x5, x6], axis=-1)


# speedup vs baseline: 1.1983x; 1.1182x over previous
import functools

import jax
import jax.numpy as jnp
from jax import lax
from jax.experimental import pallas as pl
from jax.experimental.pallas import tpu as pltpu
from jax.experimental.pallas import tpu_sc as plsc

N = 10000
E = 320000
DIM = 128
NC = 2
NS = 16
CH = 80
EPT = E // NS
NCHUNK = EPT // CH
HN = 5120
AGGR = HN + 8
RPT = HN // NS
ZR = 16
NBUF = 2


def _sc_agg_body(x_hbm, e_hbm, src_hbm, dst_hbm, parts_hbm,
                 si0, si1, di0, di1, g0, g1, e0, e1, d20, d21, d30, d31,
                 z_v, agg_sh,
                 isem0, isem1, gsem0, gsem1, esem0, esem1, ssem0, ssem1):
    c = lax.axis_index("c")
    s = lax.axis_index("s")
    si = (si0, si1)
    di = (di0, di1)
    g = (g0, g1)
    eb = (e0, e1)
    d2 = (d20, d21)
    d3 = (d30, d31)
    isem = (isem0, isem1)
    gsem = (gsem0, gsem1)
    esem = (esem0, esem1)
    ssem = (ssem0, ssem1)

    def zrow(r, carry):
        for v in range(DIM // 16):
            z_v[r, pl.ds(v * 16, 16)] = jnp.zeros((16,), jnp.float32)
        return carry
    lax.fori_loop(0, ZR, zrow, 0)
    for k in range(RPT // ZR):
        pltpu.sync_copy(z_v, agg_sh.at[pl.ds(s * RPT + k * ZR, ZR)])
    plsc.subcore_barrier()

    ebase = s * EPT
    nbase = c * HN

    def load_idx(j, b):
        pltpu.async_copy(src_hbm.at[pl.ds(ebase + j * CH, CH)],
                         si[b].at[0], isem[b])
        pltpu.async_copy(dst_hbm.at[pl.ds(ebase + j * CH, CH)],
                         di[b].at[0], isem[b])

    def wait_idx(j, b):
        pltpu.make_async_copy(src_hbm.at[pl.ds(ebase + j * CH, CH)],
                              si[b].at[0], isem[b]).wait()
        pltpu.make_async_copy(dst_hbm.at[pl.ds(ebase + j * CH, CH)],
                              di[b].at[0], isem[b]).wait()

    def remap(b):
        for v in range(CH // 16):
            sl = pl.ds(v * 16, 16)
            t = di[b][0, sl] - nbase
            ok = (t >= 0) & (t < HN)
            d2[b][0, sl] = jnp.where(ok, t, -1)
            d3[b][0, sl] = jnp.where(ok, si[b][0, sl], -1)

    def load_data(j, b):
        pltpu.async_copy(x_hbm.at[plsc.Indices(d3[b].at[0], ignored_value=-1)],
                         g[b], gsem[b])
        pltpu.async_copy(e_hbm.at[pl.ds(ebase + j * CH, CH)], eb[b], esem[b])

    def wait_data(j, b):
        pltpu.make_async_copy(
            x_hbm.at[plsc.Indices(d3[b].at[0], ignored_value=-1)],
            g[b], gsem[b]).wait()
        pltpu.make_async_copy(
            e_hbm.at[pl.ds(ebase + j * CH, CH)], eb[b], esem[b]).wait()

    def wait_scatter(b):
        pltpu.make_async_copy(
            g[b], agg_sh.at[plsc.Indices(d2[b].at[0], ignored_value=-1)],
            ssem[b]).wait()

    load_idx(0, 0)
    load_idx(1, 1)
    wait_idx(0, 0)
    remap(0)
    load_data(0, 0)

    def step(i, carry):
        j2 = i * NBUF
        for b in range(NBUF):
            j = j2 + b
            o = 1 - b
            wait_data(j, b)

            @pl.when(j + 1 < NCHUNK)
            def _():
                @pl.when(j >= 1)
                def _():
                    wait_scatter(o)
                wait_idx(j + 1, o)
                remap(o)
                load_data(j + 1, o)

            @plsc.parallel_loop(0, CH, step=1, unroll=4)
            def crow(r):
                for v in range(DIM // 16):
                    sl = pl.ds(v * 16, 16)
                    g[b][r, sl] = jnp.maximum(g[b][r, sl] + eb[b][r, sl], 0.0)

            pltpu.async_copy(
                g[b],
                agg_sh.at[plsc.Indices(d2[b].at[0], ignored_value=-1)],
                ssem[b], add=True)

            @pl.when(j + NBUF < NCHUNK)
            def _():
                load_idx(j + NBUF, b)
        return carry
    lax.fori_loop(0, NCHUNK // NBUF, step, 0)
    for b in range(NBUF):
        wait_scatter(b)

    plsc.subcore_barrier()
    pltpu.sync_copy(agg_sh.at[pl.ds(s * RPT, RPT)],
                    parts_hbm.at[c, pl.ds(s * RPT, RPT)])


_sc_agg = pl.kernel(
    _sc_agg_body,
    out_type=jax.ShapeDtypeStruct((NC, HN, DIM), jnp.float32),
    mesh=plsc.VectorSubcoreMesh(core_axis_name="c", subcore_axis_name="s"),
    scratch_types=[
        pltpu.VMEM((8, CH), jnp.int32),
        pltpu.VMEM((8, CH), jnp.int32),
        pltpu.VMEM((8, CH), jnp.int32),
        pltpu.VMEM((8, CH), jnp.int32),
        pltpu.VMEM((CH, DIM), jnp.float32),
        pltpu.VMEM((CH, DIM), jnp.float32),
        pltpu.VMEM((CH, DIM), jnp.float32),
        pltpu.VMEM((CH, DIM), jnp.float32),
        pltpu.VMEM((8, CH), jnp.int32),
        pltpu.VMEM((8, CH), jnp.int32),
        pltpu.VMEM((8, CH), jnp.int32),
        pltpu.VMEM((8, CH), jnp.int32),
        pltpu.VMEM((ZR, DIM), jnp.float32),
        pltpu.VMEM_SHARED((AGGR, DIM), jnp.float32),
        pltpu.SemaphoreType.DMA,
        pltpu.SemaphoreType.DMA,
        pltpu.SemaphoreType.DMA,
        pltpu.SemaphoreType.DMA,
        pltpu.SemaphoreType.DMA,
        pltpu.SemaphoreType.DMA,
        pltpu.SemaphoreType.DMA,
        pltpu.SemaphoreType.DMA,
    ],
)


def _bnx_body(x_ref, xi_ref, g_ref, b_ref, o_ref):
    x = x_ref[...] * xi_ref[...]
    m = jnp.mean(x, axis=0, keepdims=True)
    v = jnp.mean(jnp.square(x - m), axis=0, keepdims=True)
    o_ref[...] = (x - m) * lax.rsqrt(v + 1e-5) * g_ref[...] + b_ref[...]


def _bnx(X, Xi, g, b):
    return pl.pallas_call(
        _bnx_body,
        out_shape=jax.ShapeDtypeStruct((N, DIM), jnp.float32),
    )(X, Xi, g.reshape(1, DIM), b.reshape(1, DIM))


def _estats_body(e_ref, o_ref):
    i = pl.program_id(0)
    x = e_ref[...]
    part = jnp.concatenate(
        [jnp.sum(x, axis=0, keepdims=True),
         jnp.sum(jnp.square(x), axis=0, keepdims=True)], axis=0)

    @pl.when(i == 0)
    def _():
        o_ref[...] = part

    @pl.when(i > 0)
    def _():
        o_ref[...] += part


def _estats(e_flat, block=8000):
    n = e_flat.shape[0]
    return pl.pallas_call(
        _estats_body,
        grid=(n // block,),
        in_specs=[pl.BlockSpec((block, DIM), lambda i: (i, 0))],
        out_specs=pl.BlockSpec((2, DIM), lambda i: (0, 0)),
        out_shape=jax.ShapeDtypeStruct((2, DIM), jnp.float32),
    )(e_flat)


def _mlp_body(x_ref, w_ref, b_ref, o_ref):
    acc = jnp.dot(x_ref[...], w_ref[...], preferred_element_type=jnp.float32)
    o_ref[...] = jnp.tanh(acc + b_ref[...])


def _mlp(x, Wt, b, block):
    n, d_in = x.shape
    d_out = Wt.shape[1]
    return pl.pallas_call(
        _mlp_body,
        grid=(n // block,),
        in_specs=[
            pl.BlockSpec((block, d_in), lambda i: (i, 0)),
            pl.BlockSpec((d_in, d_out), lambda i: (0, 0)),
            pl.BlockSpec((1, d_out), lambda i: (0, 0)),
        ],
        out_specs=pl.BlockSpec((block, d_out), lambda i: (i, 0)),
        out_shape=jax.ShapeDtypeStruct((n, d_out), jnp.float32),
    )(x, Wt, b.reshape(1, -1))


def _node_body(x_ref, p_ref, w_ref, b_ref, o_ref):
    sm = x_ref[...] + p_ref[...]
    acc = jnp.dot(sm, w_ref[...], preferred_element_type=jnp.float32)
    o_ref[...] = jnp.tanh(acc + b_ref[...])


def _node_update(x, parts_flat, W, b, block=2000):
    return pl.pallas_call(
        _node_body,
        grid=(N // block,),
        in_specs=[
            pl.BlockSpec((block, DIM), lambda i: (i, 0)),
            pl.BlockSpec((block, DIM), lambda i: (i, 0)),
            pl.BlockSpec((DIM, DIM), lambda i: (0, 0)),
            pl.BlockSpec((1, DIM), lambda i: (0, 0)),
        ],
        out_specs=pl.BlockSpec((block, DIM), lambda i: (i, 0)),
        out_shape=jax.ShapeDtypeStruct((N, DIM), jnp.float32),
    )(x, parts_flat, W.T, b.reshape(1, -1))


def kernel(X, X_importance, edge_index, edge_attr, g_in, b_in, g_e, b_e,
           We1, be1, W1, b1, We2, be2, W2, b2, We3, be3, W3, b3,
           We4, be4, W4, b4, We5, be5, W5, b5, Wfc):
    srcM = edge_index[0]
    dstM = edge_index[1]

    x = _bnx(X, X_importance, g_in, b_in)

    e_flat = edge_attr.reshape(E * 16 // DIM, DIM)
    st = _estats(e_flat)
    s8 = st[0].reshape(DIM // 16, 16).sum(axis=0)
    ss8 = st[1].reshape(DIM // 16, 16).sum(axis=0)
    mean = s8 / E
    var = ss8 / E - jnp.square(mean)
    inv = lax.rsqrt(var + 1e-5)
    scale = g_e * inv
    shift = b_e - mean * scale
    Wp = We1.T * scale[:, None]
    bp = be1 + shift @ We1.T

    e = _mlp(edge_attr, Wp, bp, block=4000)

    def gine(x, e, W, b):
        parts = _sc_agg(x, e, srcM, dstM)
        agg = parts.reshape(NC * HN, DIM)
        return _node_update(x, agg, W, b)

    x1 = gine(x, e, W1, b1)
    e = _mlp(e, We2.T, be2, block=4000)
    x2 = gine(x1, e, W2, b2)
    e = _mlp(e, We3.T, be3, block=4000)
    x3 = gine(x2, e, W3, b3)
    e = _mlp(e, We4.T, be4, block=4000)
    x4 = gine(x3, e, W4, b4)
    e = _mlp(e, We5.T, be5, block=4000)
    x5 = gine(x4, e, W5, b5)
    x6 = _mlp(x5, Wfc.T, jnp.zeros((DIM,), jnp.float32), block=2000)
    return jnp.concatenate([x1, x2, x3, x4, x5, x6], axis=-1)


# speedup vs baseline: 1.2645x; 1.0553x over previous
import functools

import jax
import jax.numpy as jnp
from jax import lax
from jax.experimental import pallas as pl
from jax.experimental.pallas import tpu as pltpu
from jax.experimental.pallas import tpu_sc as plsc

N = 10000
E = 320000
DIM = 128
NC = 2
NS = 16
CH = 80
EPT = E // NS
NCHUNK = EPT // CH
HN = 5120
AGGR = HN + 8
RPT = HN // NS
ZR = 16
NBUF = 2


def _sc_agg_body(x_hbm, e_hbm, src_hbm, dst_hbm, parts_hbm,
                 si0, si1, di0, di1, g0, g1, e0, e1, d20, d21, d30, d31,
                 d40, d41, z_v, agg_sh,
                 isem0, isem1, gsem0, gsem1, esem0, esem1, ssem0, ssem1):
    c = lax.axis_index("c")
    s = lax.axis_index("s")
    si = (si0, si1)
    di = (di0, di1)
    g = (g0, g1)
    eb = (e0, e1)
    d2 = (d20, d21)
    d3 = (d30, d31)
    d4 = (d40, d41)
    isem = (isem0, isem1)
    gsem = (gsem0, gsem1)
    esem = (esem0, esem1)
    ssem = (ssem0, ssem1)

    def zrow(r, carry):
        for v in range(DIM // 16):
            z_v[r, pl.ds(v * 16, 16)] = jnp.zeros((16,), jnp.float32)
        return carry
    lax.fori_loop(0, ZR, zrow, 0)
    for k in range(RPT // ZR):
        pltpu.sync_copy(z_v, agg_sh.at[pl.ds(s * RPT + k * ZR, ZR)])
    plsc.subcore_barrier()

    ebase = s * EPT
    nbase = c * HN

    def load_idx(j, b):
        pltpu.async_copy(src_hbm.at[pl.ds(ebase + j * CH, CH)],
                         si[b].at[0], isem[b])
        pltpu.async_copy(dst_hbm.at[pl.ds(ebase + j * CH, CH)],
                         di[b].at[0], isem[b])

    def wait_idx(j, b):
        pltpu.make_async_copy(src_hbm.at[pl.ds(ebase + j * CH, CH)],
                              si[b].at[0], isem[b]).wait()
        pltpu.make_async_copy(dst_hbm.at[pl.ds(ebase + j * CH, CH)],
                              di[b].at[0], isem[b]).wait()

    def remap(j, b):
        for v in range(CH // 16):
            sl = pl.ds(v * 16, 16)
            t = di[b][0, sl] - nbase
            ok = (t >= 0) & (t < HN)
            d2[b][0, sl] = jnp.where(ok, t, -1)
            d3[b][0, sl] = jnp.where(ok, si[b][0, sl], -1)
            eidx = ebase + j * CH + v * 16 + lax.iota(jnp.int32, 16)
            d4[b][0, sl] = jnp.where(ok, eidx, -1)

    def load_data(j, b):
        pltpu.async_copy(x_hbm.at[plsc.Indices(d3[b].at[0], ignored_value=-1)],
                         g[b], gsem[b])
        pltpu.async_copy(e_hbm.at[plsc.Indices(d4[b].at[0], ignored_value=-1)],
                         eb[b], esem[b])

    def wait_data(j, b):
        pltpu.make_async_copy(
            x_hbm.at[plsc.Indices(d3[b].at[0], ignored_value=-1)],
            g[b], gsem[b]).wait()
        pltpu.make_async_copy(
            e_hbm.at[plsc.Indices(d4[b].at[0], ignored_value=-1)],
            eb[b], esem[b]).wait()

    def wait_scatter(b):
        pltpu.make_async_copy(
            g[b], agg_sh.at[plsc.Indices(d2[b].at[0], ignored_value=-1)],
            ssem[b]).wait()

    load_idx(0, 0)
    load_idx(1, 1)
    wait_idx(0, 0)
    remap(0, 0)
    load_data(0, 0)

    def step(i, carry):
        j2 = i * NBUF
        for b in range(NBUF):
            j = j2 + b
            o = 1 - b
            wait_data(j, b)

            @pl.when(j + 1 < NCHUNK)
            def _():
                @pl.when(j >= 1)
                def _():
                    wait_scatter(o)
                wait_idx(j + 1, o)
                remap(j + 1, o)
                load_data(j + 1, o)

            @plsc.parallel_loop(0, CH, step=1, unroll=4)
            def crow(r):
                for v in range(DIM // 16):
                    sl = pl.ds(v * 16, 16)
                    g[b][r, sl] = jnp.maximum(g[b][r, sl] + eb[b][r, sl], 0.0)

            pltpu.async_copy(
                g[b],
                agg_sh.at[plsc.Indices(d2[b].at[0], ignored_value=-1)],
                ssem[b], add=True)

            @pl.when(j + NBUF < NCHUNK)
            def _():
                load_idx(j + NBUF, b)
        return carry
    lax.fori_loop(0, NCHUNK // NBUF, step, 0)
    for b in range(NBUF):
        wait_scatter(b)

    plsc.subcore_barrier()
    pltpu.sync_copy(agg_sh.at[pl.ds(s * RPT, RPT)],
                    parts_hbm.at[c, pl.ds(s * RPT, RPT)])


_sc_agg = pl.kernel(
    _sc_agg_body,
    out_type=jax.ShapeDtypeStruct((NC, HN, DIM), jnp.float32),
    mesh=plsc.VectorSubcoreMesh(core_axis_name="c", subcore_axis_name="s"),
    scratch_types=[
        pltpu.VMEM((8, CH), jnp.int32),
        pltpu.VMEM((8, CH), jnp.int32),
        pltpu.VMEM((8, CH), jnp.int32),
        pltpu.VMEM((8, CH), jnp.int32),
        pltpu.VMEM((CH, DIM), jnp.float32),
        pltpu.VMEM((CH, DIM), jnp.float32),
        pltpu.VMEM((CH, DIM), jnp.float32),
        pltpu.VMEM((CH, DIM), jnp.float32),
        pltpu.VMEM((8, CH), jnp.int32),
        pltpu.VMEM((8, CH), jnp.int32),
        pltpu.VMEM((8, CH), jnp.int32),
        pltpu.VMEM((8, CH), jnp.int32),
        pltpu.VMEM((8, CH), jnp.int32),
        pltpu.VMEM((8, CH), jnp.int32),
        pltpu.VMEM((ZR, DIM), jnp.float32),
        pltpu.VMEM_SHARED((AGGR, DIM), jnp.float32),
        pltpu.SemaphoreType.DMA,
        pltpu.SemaphoreType.DMA,
        pltpu.SemaphoreType.DMA,
        pltpu.SemaphoreType.DMA,
        pltpu.SemaphoreType.DMA,
        pltpu.SemaphoreType.DMA,
        pltpu.SemaphoreType.DMA,
        pltpu.SemaphoreType.DMA,
    ],
)


def _bnx_body(x_ref, xi_ref, g_ref, b_ref, o_ref):
    x = x_ref[...] * xi_ref[...]
    m = jnp.mean(x, axis=0, keepdims=True)
    v = jnp.mean(jnp.square(x - m), axis=0, keepdims=True)
    o_ref[...] = (x - m) * lax.rsqrt(v + 1e-5) * g_ref[...] + b_ref[...]


def _bnx(X, Xi, g, b):
    return pl.pallas_call(
        _bnx_body,
        out_shape=jax.ShapeDtypeStruct((N, DIM), jnp.float32),
    )(X, Xi, g.reshape(1, DIM), b.reshape(1, DIM))


def _estats_body(e_ref, o_ref):
    i = pl.program_id(0)
    x = e_ref[...]
    part = jnp.concatenate(
        [jnp.sum(x, axis=0, keepdims=True),
         jnp.sum(jnp.square(x), axis=0, keepdims=True)], axis=0)

    @pl.when(i == 0)
    def _():
        o_ref[...] = part

    @pl.when(i > 0)
    def _():
        o_ref[...] += part


def _estats(e_flat, block=8000):
    n = e_flat.shape[0]
    return pl.pallas_call(
        _estats_body,
        grid=(n // block,),
        in_specs=[pl.BlockSpec((block, DIM), lambda i: (i, 0))],
        out_specs=pl.BlockSpec((2, DIM), lambda i: (0, 0)),
        out_shape=jax.ShapeDtypeStruct((2, DIM), jnp.float32),
    )(e_flat)


def _mlp_body(x_ref, w_ref, b_ref, o_ref):
    acc = jnp.dot(x_ref[...], w_ref[...], preferred_element_type=jnp.float32)
    o_ref[...] = jnp.tanh(acc + b_ref[...])


def _mlp(x, Wt, b, block):
    n, d_in = x.shape
    d_out = Wt.shape[1]
    return pl.pallas_call(
        _mlp_body,
        grid=(n // block,),
        in_specs=[
            pl.BlockSpec((block, d_in), lambda i: (i, 0)),
            pl.BlockSpec((d_in, d_out), lambda i: (0, 0)),
            pl.BlockSpec((1, d_out), lambda i: (0, 0)),
        ],
        out_specs=pl.BlockSpec((block, d_out), lambda i: (i, 0)),
        out_shape=jax.ShapeDtypeStruct((n, d_out), jnp.float32),
    )(x, Wt, b.reshape(1, -1))


def _node_body(x_ref, p_ref, w_ref, b_ref, o_ref):
    sm = x_ref[...] + p_ref[...]
    acc = jnp.dot(sm, w_ref[...], preferred_element_type=jnp.float32)
    o_ref[...] = jnp.tanh(acc + b_ref[...])


def _node_update(x, parts_flat, W, b, block=2000):
    return pl.pallas_call(
        _node_body,
        grid=(N // block,),
        in_specs=[
            pl.BlockSpec((block, DIM), lambda i: (i, 0)),
            pl.BlockSpec((block, DIM), lambda i: (i, 0)),
            pl.BlockSpec((DIM, DIM), lambda i: (0, 0)),
            pl.BlockSpec((1, DIM), lambda i: (0, 0)),
        ],
        out_specs=pl.BlockSpec((block, DIM), lambda i: (i, 0)),
        out_shape=jax.ShapeDtypeStruct((N, DIM), jnp.float32),
    )(x, parts_flat, W.T, b.reshape(1, -1))


def kernel(X, X_importance, edge_index, edge_attr, g_in, b_in, g_e, b_e,
           We1, be1, W1, b1, We2, be2, W2, b2, We3, be3, W3, b3,
           We4, be4, W4, b4, We5, be5, W5, b5, Wfc):
    srcM = edge_index[0]
    dstM = edge_index[1]

    x = _bnx(X, X_importance, g_in, b_in)

    e_flat = edge_attr.reshape(E * 16 // DIM, DIM)
    st = _estats(e_flat)
    s8 = st[0].reshape(DIM // 16, 16).sum(axis=0)
    ss8 = st[1].reshape(DIM // 16, 16).sum(axis=0)
    mean = s8 / E
    var = ss8 / E - jnp.square(mean)
    inv = lax.rsqrt(var + 1e-5)
    scale = g_e * inv
    shift = b_e - mean * scale
    Wp = We1.T * scale[:, None]
    bp = be1 + shift @ We1.T

    e = _mlp(edge_attr, Wp, bp, block=4000)

    def gine(x, e, W, b):
        parts = _sc_agg(x, e, srcM, dstM)
        agg = parts.reshape(NC * HN, DIM)
        return _node_update(x, agg, W, b)

    x1 = gine(x, e, W1, b1)
    e = _mlp(e, We2.T, be2, block=4000)
    x2 = gine(x1, e, W2, b2)
    e = _mlp(e, We3.T, be3, block=4000)
    x3 = gine(x2, e, W3, b3)
    e = _mlp(e, We4.T, be4, block=4000)
    x4 = gine(x3, e, W4, b4)
    e = _mlp(e, We5.T, be5, block=4000)
    x5 = gine(x4, e, W5, b5)
    x6 = _mlp(x5, Wfc.T, jnp.zeros((DIM,), jnp.float32), block=2000)
    return jnp.concatenate([x1, x2, x3, x4, x5, x6], axis=-1)
